# speedup vs baseline: 1.0059x; 1.0059x over previous
import jax
import jax.numpy as jnp
from jax.experimental import pallas as pl


def _mlp_kernel(gx_ref, w1_ref, b1_ref, w2_ref, b2_ref, out_ref):
    gx = gx_ref[...]
    h = jnp.maximum(gx @ w1_ref[...] + b1_ref[...][None, :], 0.0)
    out_ref[...] = h @ w2_ref[...] + b2_ref[...][None, :]


def kernel(x, edge_index, edge_attr, batch, lin0_W, lin0_b, Wrel0, brel0, Wroot0, gamma0, beta0, Wrel1, brel1, Wroot1, gamma1, beta1, lin1_W, lin1_b, lin2_W, lin2_b):
    G = 512
    edge_weight = edge_attr.reshape(-1)
    src = edge_index[0]
    dst = edge_index[1]

    h = jnp.tanh(x @ lin0_W + lin0_b)

    def conv(h, Wrel, brel, Wroot):
        msgs = jnp.take(h, src, axis=0) * edge_weight[:, None]
        agg = jnp.zeros_like(h).at[dst].add(msgs)
        return agg @ Wrel + brel + h @ Wroot

    def bn(z, gamma, beta, eps=1e-5):
        mu = jnp.mean(z, axis=0)
        var = jnp.var(z, axis=0)
        return gamma * (z - mu) / jnp.sqrt(var + eps) + beta

    h = bn(jax.nn.relu(conv(h, Wrel0, brel0, Wroot0)), gamma0, beta0)
    h = bn(jax.nn.relu(conv(h, Wrel1, brel1, Wroot1)), gamma1, beta1)

    sums = jax.ops.segment_sum(h, batch, num_segments=G)
    counts = jax.ops.segment_sum(jnp.ones((h.shape[0], 1), h.dtype), batch, num_segments=G)
    gx = sums / jnp.clip(counts, 1.0, None)

    return pl.pallas_call(
        _mlp_kernel,
        out_shape=jax.ShapeDtypeStruct((G, lin2_W.shape[1]), jnp.float32),
    )(gx, lin1_W, lin1_b, lin2_W, lin2_b)


# speedup vs baseline: 8.9013x; 8.8494x over previous
import functools

import jax
import jax.numpy as jnp
from jax import lax
from jax.experimental import pallas as pl
from jax.experimental.pallas import tpu as pltpu
from jax.experimental.pallas import tpu_sc as plsc

N = 100000
E = 1600000
G = 512
H = 32
HH = 16
EPS = 1e-5

EPAD = 1638400
ROWS = EPAD // 128
RPT = ROWS // 16
CHUNKS = 100
CROWS = 8
CE = CROWS * 128

ACC_SPAN = 6256
ACC_LAST = N - 15 * ACC_SPAN
W_SPAN = 6400
W_LAST = N - 15 * W_SPAN


def _zero_rows(buf, nrows):
    def body(i, _):
        buf[i, :] = jnp.zeros((16,), jnp.float32)
        return 0
    lax.fori_loop(0, nrows, body, 0)


def _zero_1d(buf, n):
    def body(i, _):
        buf[pl.ds(i * 16, 16)] = jnp.zeros((16,), jnp.float32)
        return 0
    lax.fori_loop(0, n // 16, body, 0)


def _copy_span(src_buf, dst_ref, base, total):
    off = 0
    while off < total:
        sz = min(CE, total - off)
        pltpu.sync_copy(src_buf.at[pl.ds(0, sz)], dst_ref.at[pl.ds(base + off, sz)])
        off += sz


def _copy_span_1d(src_buf, dst_ref, base, total):
    off = 0
    while off < total:
        sz = min(2048, total - off)
        pltpu.sync_copy(src_buf.at[pl.ds(0, sz)], dst_ref.at[pl.ds(base + off, sz)])
        off += sz


def _agg_pipeline(s, src_hbm, dst_hbm, w_hbm, tab, out_hbm, acc,
                  src_v, dst_v, w_v, msgs, sem, do_wsum, accw, wsum_hbm, zw1):
    _zero_rows(msgs, CE)
    for t in range(16):
        span = ACC_SPAN if t < 15 else ACC_LAST

        @pl.when(s == t)
        def _(t=t, span=span):
            _copy_span(msgs, acc, t * ACC_SPAN, span)

    if do_wsum:
        _zero_1d(zw1, 2048)
        for t in range(16):
            span = W_SPAN if t < 15 else W_LAST

            @pl.when(s == t)
            def _(t=t, span=span):
                _copy_span_1d(zw1, accw, t * W_SPAN, span)

    plsc.subcore_barrier()

    def chunk_body(i, _):
        row0 = s * RPT + i * CROWS
        pltpu.sync_copy(src_hbm.at[pl.ds(row0, CROWS)], src_v)
        pltpu.sync_copy(dst_hbm.at[pl.ds(row0, CROWS)], dst_v)
        pltpu.sync_copy(w_hbm.at[pl.ds(row0 * 128, CE)], w_v)
        descs = [pltpu.async_copy(tab.at[src_v.at[j]],
                                  msgs.at[pl.ds(j * 128, 128)], sem)
                 for j in range(CROWS)]
        for dsc in descs:
            dsc.wait()

        def mul(g, _):
            wvec = w_v[pl.ds(g * 16, 16)]
            base = g * 16
            for u in range(16):
                msgs[base + u, :] = msgs[base + u, :] * wvec[u]
            return 0
        lax.fori_loop(0, CE // 16, mul, 0)

        for j in range(CROWS):
            pltpu.sync_copy(msgs.at[pl.ds(j * 128, 128)],
                            acc.at[dst_v.at[j]], add=True)
        if do_wsum:
            for j in range(CROWS):
                pltpu.sync_copy(w_v.at[pl.ds(j * 128, 128)],
                                accw.at[dst_v.at[j]], add=True)
        return 0

    lax.fori_loop(0, CHUNKS, chunk_body, 0)

    plsc.subcore_barrier()

    for t in range(16):
        span = ACC_SPAN if t < 15 else ACC_LAST

        @pl.when(s == t)
        def _(t=t, span=span):
            pltpu.sync_copy(acc.at[pl.ds(t * ACC_SPAN, span)],
                            out_hbm.at[pl.ds(t * ACC_SPAN, span)])

    if do_wsum:
        for t in range(16):
            span = W_SPAN if t < 15 else W_LAST

            @pl.when(s == t)
            def _(t=t, span=span):
                pltpu.sync_copy(accw.at[pl.ds(t * W_SPAN, span)],
                                wsum_hbm.at[pl.ds(t * W_SPAN, span)])


def _make_sc_agg(with_wsum):
    info = plsc.get_sparse_core_info()
    mesh = plsc.VectorSubcoreMesh(core_axis_name="c", subcore_axis_name="s")

    out_type = [jax.ShapeDtypeStruct((N, HH), jnp.float32),
                jax.ShapeDtypeStruct((N, HH), jnp.float32)]
    if with_wsum:
        out_type.append(jax.ShapeDtypeStruct((N,), jnp.float32))

    scratch = [
        pltpu.VMEM_SHARED((15 * ACC_SPAN + ACC_LAST + 96, HH), jnp.float32),
        pltpu.VMEM((CROWS, 128), jnp.int32),
        pltpu.VMEM((CROWS, 128), jnp.int32),
        pltpu.VMEM((CE,), jnp.float32),
        pltpu.VMEM((CE, HH), jnp.float32),
        pltpu.SemaphoreType.DMA,
    ]
    if with_wsum:
        scratch += [
            pltpu.VMEM_SHARED((15 * W_SPAN + W_LAST + 2400,), jnp.float32),
            pltpu.VMEM((2048,), jnp.float32),
        ]

    def body(src_hbm, dst_hbm, w_hbm, tab_lo, tab_hi, *rest):
        if with_wsum:
            out_lo, out_hi, wsum_hbm, acc, src_v, dst_v, w_v, msgs, sem, accw, zw1 = rest
        else:
            out_lo, out_hi, acc, src_v, dst_v, w_v, msgs, sem = rest
            wsum_hbm = accw = zw1 = None
        c = lax.axis_index("c")
        s = lax.axis_index("s")

        @pl.when(c == 0)
        def _():
            _agg_pipeline(s, src_hbm, dst_hbm, w_hbm, tab_lo, out_lo, acc,
                          src_v, dst_v, w_v, msgs, sem, with_wsum, accw,
                          wsum_hbm, zw1)

        @pl.when(c == 1)
        def _():
            _agg_pipeline(s, src_hbm, dst_hbm, w_hbm, tab_hi, out_hi, acc,
                          src_v, dst_v, w_v, msgs, sem, False, None, None, None)

    return pl.kernel(body, out_type=tuple(out_type), mesh=mesh,
                     scratch_types=scratch,
                     compiler_params=pltpu.CompilerParams(
                         use_tc_tiling_on_sc=False))


_sc_agg_wsum = _make_sc_agg(True)
_sc_agg = _make_sc_agg(False)


def _bdot(a, b):
    return jnp.dot(a.astype(jnp.bfloat16), b.astype(jnp.bfloat16),
                   preferred_element_type=jnp.float32)


BL1 = 8192
BL3 = 4096
BL5 = 2048


def _k1_body(x_ref, w_ref, b_ref, lo_ref, hi_ref):
    h = jnp.tanh(x_ref[...] * w_ref[...] + b_ref[...])
    lo_ref[...] = h[:, :HH]
    hi_ref[...] = h[:, HH:]


def _k1(x, lin0_W, lin0_b):
    grid = (pl.cdiv(N, BL1),)
    return pl.pallas_call(
        _k1_body,
        grid=grid,
        in_specs=[
            pl.BlockSpec((BL1, 1), lambda i: (i, 0)),
            pl.BlockSpec((1, H), lambda i: (0, 0)),
            pl.BlockSpec((1, H), lambda i: (0, 0)),
        ],
        out_specs=[
            pl.BlockSpec((BL1, HH), lambda i: (i, 0)),
            pl.BlockSpec((BL1, HH), lambda i: (i, 0)),
        ],
        out_shape=[jax.ShapeDtypeStruct((N, HH), jnp.float32),
                   jax.ShapeDtypeStruct((N, HH), jnp.float32)],
    )(x, lin0_W, lin0_b)


def _k3_body(alo_ref, ahi_ref, hlo_ref, hhi_ref, wrel_ref, brel_ref,
             wroot_ref, gamma_ref, beta_ref,
             zlo_ref, zhi_ref, s_ref, t_ref, st_acc):
    pid = pl.program_id(0)
    nb = pl.num_programs(0)
    agg = jnp.concatenate([alo_ref[...], ahi_ref[...]], axis=1)
    h = jnp.concatenate([hlo_ref[...], hhi_ref[...]], axis=1)
    z = jnp.maximum(_bdot(agg, wrel_ref[...]) + brel_ref[...]
                    + _bdot(h, wroot_ref[...]), 0.0)
    row = pid * BL3 + lax.broadcasted_iota(jnp.int32, (BL3, 1), 0)
    valid = (row < N).astype(jnp.float32)
    zm = z * valid
    zlo_ref[...] = z[:, :HH]
    zhi_ref[...] = z[:, HH:]
    ssum = jnp.sum(zm, axis=0).reshape(1, H)
    ssq = jnp.sum(zm * zm, axis=0).reshape(1, H)

    @pl.when(pid == 0)
    def _():
        st_acc[0:1, :] = ssum
        st_acc[1:2, :] = ssq

    @pl.when(pid > 0)
    def _():
        st_acc[0:1, :] = st_acc[0:1, :] + ssum
        st_acc[1:2, :] = st_acc[1:2, :] + ssq

    @pl.when(pid == nb - 1)
    def _():
        mu = st_acc[0:1, :] / float(N)
        var = st_acc[1:2, :] / float(N) - mu * mu
        s = gamma_ref[...] / jnp.sqrt(var + EPS)
        s_ref[...] = s
        t_ref[...] = beta_ref[...] - mu * s


def _k3(agg_lo, agg_hi, h_lo, h_hi, Wrel, brel, Wroot, gamma, beta):
    grid = (pl.cdiv(N, BL3),)
    full = lambda r, c: pl.BlockSpec((r, c), lambda i: (0, 0))
    blk = lambda: pl.BlockSpec((BL3, HH), lambda i: (i, 0))
    return pl.pallas_call(
        _k3_body,
        grid=grid,
        in_specs=[blk(), blk(), blk(), blk(),
                  full(H, H), full(1, H), full(H, H), full(1, H), full(1, H)],
        out_specs=[blk(), blk(), full(1, H), full(1, H)],
        out_shape=[jax.ShapeDtypeStruct((N, HH), jnp.float32),
                   jax.ShapeDtypeStruct((N, HH), jnp.float32),
                   jax.ShapeDtypeStruct((1, H), jnp.float32),
                   jax.ShapeDtypeStruct((1, H), jnp.float32)],
        scratch_shapes=[pltpu.VMEM((8, H), jnp.float32)],
    )(agg_lo, agg_hi, h_lo, h_hi, Wrel, brel, Wroot, gamma, beta)


def _k5_body(alo_ref, ahi_ref, zlo_ref, zhi_ref, ws_ref, batch_ref,
             s0_ref, t0_ref, wrel_ref, brel_ref, wroot_ref, gamma_ref, beta_ref,
             w1_ref, b1_ref, w2_ref, b2_ref,
             out_ref, st_acc, pool_acc, cnt_acc):
    pid = pl.program_id(0)
    nb = pl.num_programs(0)
    s0 = s0_ref[...]
    t0 = t0_ref[...]
    aggz = jnp.concatenate([alo_ref[...], ahi_ref[...]], axis=1)
    z0 = jnp.concatenate([zlo_ref[...], zhi_ref[...]], axis=1)
    h1 = z0 * s0 + t0
    agg1 = aggz * s0 + ws_ref[...] * t0
    z1 = jnp.maximum(_bdot(agg1, wrel_ref[...]) + brel_ref[...]
                     + _bdot(h1, wroot_ref[...]), 0.0)
    row = pid * BL5 + lax.broadcasted_iota(jnp.int32, (BL5, 1), 0)
    valid = (row < N).astype(jnp.float32)
    z1 = z1 * valid
    ssum = jnp.sum(z1, axis=0).reshape(1, H)
    ssq = jnp.sum(z1 * z1, axis=0).reshape(1, H)
    gids = lax.broadcasted_iota(jnp.int32, (BL5, G), 1)
    oh = (batch_ref[...] == gids).astype(jnp.float32) * valid
    pool = lax.dot_general(oh, z1, (((0,), (0,)), ((), ())),
                           preferred_element_type=jnp.float32,
                           precision=lax.Precision.HIGHEST)
    cnt = jnp.sum(oh, axis=0).reshape(1, G)

    @pl.when(pid == 0)
    def _():
        st_acc[0:1, :] = ssum
        st_acc[1:2, :] = ssq
        pool_acc[...] = pool
        cnt_acc[0:1, :] = cnt

    @pl.when(pid > 0)
    def _():
        st_acc[0:1, :] = st_acc[0:1, :] + ssum
        st_acc[1:2, :] = st_acc[1:2, :] + ssq
        pool_acc[...] = pool_acc[...] + pool
        cnt_acc[0:1, :] = cnt_acc[0:1, :] + cnt

    @pl.when(pid == nb - 1)
    def _():
        mu = st_acc[0:1, :] / float(N)
        var = st_acc[1:2, :] / float(N) - mu * mu
        s1 = gamma_ref[...] / jnp.sqrt(var + EPS)
        t1 = beta_ref[...] - mu * s1
        cnts = jnp.maximum(cnt_acc[0:1, :], 1.0).reshape(G, 1)
        gx = pool_acc[...] / cnts * s1 + t1
        hmid = jnp.maximum(_bdot(gx, w1_ref[...]) + b1_ref[...], 0.0)
        out_ref[...] = _bdot(hmid, w2_ref[...]) + b2_ref[...]


def _k5(agg_lo, agg_hi, z_lo, z_hi, wsum, batch2d, s0, t0,
        Wrel, brel, Wroot, gamma, beta, w1, b1, w2, b2):
    grid = (pl.cdiv(N, BL5),)
    full = lambda r, c: pl.BlockSpec((r, c), lambda i: (0, 0))
    blk = lambda w: pl.BlockSpec((BL5, w), lambda i: (i, 0))
    return pl.pallas_call(
        _k5_body,
        grid=grid,
        in_specs=[blk(HH), blk(HH), blk(HH), blk(HH), blk(1), blk(1),
                  full(1, H), full(1, H), full(H, H), full(1, H), full(H, H),
                  full(1, H), full(1, H), full(H, 64), full(1, 64),
                  full(64, 5), full(1, 5)],
        out_specs=pl.BlockSpec((G, 5), lambda i: (0, 0)),
        out_shape=jax.ShapeDtypeStruct((G, 5), jnp.float32),
        scratch_shapes=[pltpu.VMEM((8, H), jnp.float32),
                        pltpu.VMEM((G, H), jnp.float32),
                        pltpu.VMEM((8, G), jnp.float32)],
    )(agg_lo, agg_hi, z_lo, z_hi, wsum, batch2d, s0, t0,
      Wrel, brel, Wroot, gamma, beta, w1, b1, w2, b2)


def kernel(x, edge_index, edge_attr, batch, lin0_W, lin0_b, Wrel0, brel0,
           Wroot0, gamma0, beta0, Wrel1, brel1, Wroot1, gamma1, beta1,
           lin1_W, lin1_b, lin2_W, lin2_b):
    src = edge_index[0]
    dst = edge_index[1]
    w = edge_attr.reshape(-1)
    pad = EPAD - E
    padidx = jnp.arange(pad, dtype=jnp.int32) % N
    src_p = jnp.concatenate([src, padidx]).reshape(ROWS, 128)
    dst_p = jnp.concatenate([dst, padidx]).reshape(ROWS, 128)
    w_p = jnp.concatenate([w, jnp.zeros((pad,), jnp.float32)])

    h_lo, h_hi = _k1(x, lin0_W, lin0_b.reshape(1, H))
    agg_lo, agg_hi, wsum = _sc_agg_wsum(src_p, dst_p, w_p, h_lo, h_hi)
    z_lo, z_hi, s0, t0 = _k3(agg_lo, agg_hi, h_lo, h_hi,
                             Wrel0, brel0.reshape(1, H), Wroot0,
                             gamma0.reshape(1, H), beta0.reshape(1, H))
    a1_lo, a1_hi = _sc_agg(src_p, dst_p, w_p, z_lo, z_hi)
    pred = _k5(a1_lo, a1_hi, z_lo, z_hi, wsum.reshape(N, 1),
               batch.reshape(N, 1), s0, t0,
               Wrel1, brel1.reshape(1, H), Wroot1,
               gamma1.reshape(1, H), beta1.reshape(1, H),
               lin1_W, lin1_b.reshape(1, 64), lin2_W, lin2_b.reshape(1, 5))
    return pred


# speedup vs baseline: 11.4225x; 1.2832x over previous
import functools

import jax
import jax.numpy as jnp
from jax import lax
from jax.experimental import pallas as pl
from jax.experimental.pallas import tpu as pltpu
from jax.experimental.pallas import tpu_sc as plsc

N = 100000
E = 1600000
G = 512
H = 32
HH = 16
EPS = 1e-5

EPAD = 1638400
ROWS = EPAD // 128
RPT = ROWS // 16
CHUNKS = 50
CROWS = 16
CE = CROWS * 128
SUB = 512
SUBS = CE // SUB

ACC_SPAN = 6256
ACC_LAST = N - 15 * ACC_SPAN
W_SPAN = 6400
W_LAST = N - 15 * W_SPAN


def _zero_rows(buf, nrows):
    def body(i, _):
        buf[i, :] = jnp.zeros((16,), jnp.float32)
        return 0
    lax.fori_loop(0, nrows, body, 0)


def _zero_1d(buf, n):
    def body(i, _):
        buf[pl.ds(i * 16, 16)] = jnp.zeros((16,), jnp.float32)
        return 0
    lax.fori_loop(0, n // 16, body, 0)


def _copy_span(src_buf, dst_ref, base, total):
    off = 0
    while off < total:
        sz = min(SUB, total - off)
        pltpu.sync_copy(src_buf.at[pl.ds(0, sz)], dst_ref.at[pl.ds(base + off, sz)])
        off += sz


def _copy_span_1d(src_buf, dst_ref, base, total):
    off = 0
    while off < total:
        sz = min(SUB, total - off)
        pltpu.sync_copy(src_buf.at[pl.ds(0, sz)], dst_ref.at[pl.ds(base + off, sz)])
        off += sz


def _agg_pipeline(s, src_hbm, dst_hbm, w_hbm, tab, out_hbm, acc,
                  src_v, dst_v, w_v, msgs2, sems, do_wsum, accw, wsum_hbm, zw1):
    msgs0, msgs1 = msgs2
    semi, semg0, semg1, sems0, sems1, semw = sems
    _zero_rows(msgs0, SUB)
    for t in range(16):
        span = ACC_SPAN if t < 15 else ACC_LAST

        @pl.when(s == t)
        def _(t=t, span=span):
            _copy_span(msgs0, acc, t * ACC_SPAN, span)

    if do_wsum:
        _zero_1d(zw1, SUB)
        for t in range(16):
            span = W_SPAN if t < 15 else W_LAST

            @pl.when(s == t)
            def _(t=t, span=span):
                _copy_span_1d(zw1, accw, t * W_SPAN, span)

    plsc.subcore_barrier()

    bufs = (msgs0, msgs1)
    gsems = (semg0, semg1)
    ssems = (sems0, sems1)

    def chunk_body(i, _):
        row0 = s * RPT + i * CROWS
        d1 = pltpu.async_copy(src_hbm.at[pl.ds(row0, CROWS)], src_v, semi)
        d2 = pltpu.async_copy(dst_hbm.at[pl.ds(row0, CROWS)], dst_v, semi)
        d3 = pltpu.async_copy(w_hbm.at[pl.ds(row0 * 128, CE)], w_v, semi)
        d1.wait()
        d2.wait()
        d3.wait()
        gd = {}

        def fire_g(sub):
            buf, sem = bufs[sub % 2], gsems[sub % 2]
            gd[sub] = [pltpu.async_copy(tab.at[src_v.at[sub * 4 + j]],
                                        buf.at[pl.ds(j * 128, 128)], sem)
                       for j in range(4)]

        def mul(sub):
            buf = bufs[sub % 2]
            woff = sub * SUB

            def body(g, _):
                wvec = w_v[pl.ds(woff + g * 16, 16)]
                base = g * 16
                for u in range(16):
                    buf[base + u, :] = buf[base + u, :] * wvec[u]
                return 0
            lax.fori_loop(0, SUB // 16, body, 0)

        fire_g(0)
        for sub in range(SUBS):
            for dsc in gd[sub]:
                dsc.wait()
            if sub + 1 < SUBS:
                fire_g(sub + 1)
            mul(sub)
            for j in range(4):
                pltpu.sync_copy(bufs[sub % 2].at[pl.ds(j * 128, 128)],
                                acc.at[dst_v.at[sub * 4 + j]], add=True)
            if do_wsum:
                for j in range(4):
                    pltpu.sync_copy(
                        w_v.at[pl.ds((sub * 4 + j) * 128, 128)],
                        accw.at[dst_v.at[sub * 4 + j]], add=True)
        return 0

    lax.fori_loop(0, CHUNKS, chunk_body, 0)

    plsc.subcore_barrier()

    for t in range(16):
        span = ACC_SPAN if t < 15 else ACC_LAST

        @pl.when(s == t)
        def _(t=t, span=span):
            pltpu.sync_copy(acc.at[pl.ds(t * ACC_SPAN, span)],
                            out_hbm.at[pl.ds(t * ACC_SPAN, span)])

    if do_wsum:
        for t in range(16):
            span = W_SPAN if t < 15 else W_LAST

            @pl.when(s == t)
            def _(t=t, span=span):
                pltpu.sync_copy(accw.at[pl.ds(t * W_SPAN, span)],
                                wsum_hbm.at[pl.ds(t * W_SPAN, span)])


def _make_sc_agg(with_wsum):
    info = plsc.get_sparse_core_info()
    mesh = plsc.VectorSubcoreMesh(core_axis_name="c", subcore_axis_name="s")

    out_type = [jax.ShapeDtypeStruct((N, HH), jnp.float32),
                jax.ShapeDtypeStruct((N, HH), jnp.float32)]
    if with_wsum:
        out_type.append(jax.ShapeDtypeStruct((N,), jnp.float32))

    scratch = [
        pltpu.VMEM_SHARED((15 * ACC_SPAN + ACC_LAST + 96, HH), jnp.float32),
        pltpu.VMEM((CROWS, 128), jnp.int32),
        pltpu.VMEM((CROWS, 128), jnp.int32),
        pltpu.VMEM((CE,), jnp.float32),
        pltpu.VMEM((SUB, HH), jnp.float32),
        pltpu.VMEM((SUB, HH), jnp.float32),
    ] + [pltpu.SemaphoreType.DMA] * 6
    if with_wsum:
        scratch += [
            pltpu.VMEM_SHARED((15 * W_SPAN + W_LAST + 2400,), jnp.float32),
            pltpu.VMEM((SUB,), jnp.float32),
        ]

    def body(src_hbm, dst_hbm, w_hbm, tab_lo, tab_hi, *rest):
        if with_wsum:
            (out_lo, out_hi, wsum_hbm, acc, src_v, dst_v, w_v, msgs0, msgs1,
             semi, semg0, semg1, sems0, sems1, semw, accw, zw1) = rest
        else:
            (out_lo, out_hi, acc, src_v, dst_v, w_v, msgs0, msgs1,
             semi, semg0, semg1, sems0, sems1, semw) = rest
            wsum_hbm = accw = zw1 = None
        sems = (semi, semg0, semg1, sems0, sems1, semw)
        c = lax.axis_index("c")
        s = lax.axis_index("s")

        @pl.when(c == 0)
        def _():
            _agg_pipeline(s, src_hbm, dst_hbm, w_hbm, tab_lo, out_lo, acc,
                          src_v, dst_v, w_v, (msgs0, msgs1), sems, with_wsum,
                          accw, wsum_hbm, zw1)

        @pl.when(c == 1)
        def _():
            _agg_pipeline(s, src_hbm, dst_hbm, w_hbm, tab_hi, out_hi, acc,
                          src_v, dst_v, w_v, (msgs0, msgs1), sems, False,
                          None, None, None)

    return pl.kernel(body, out_type=tuple(out_type), mesh=mesh,
                     scratch_types=scratch,
                     compiler_params=pltpu.CompilerParams(
                         use_tc_tiling_on_sc=False))


_sc_agg_wsum = _make_sc_agg(True)
_sc_agg = _make_sc_agg(False)


def _bdot(a, b):
    return jnp.dot(a.astype(jnp.bfloat16), b.astype(jnp.bfloat16),
                   preferred_element_type=jnp.float32)


BL1 = 8192
BL3 = 4096
BL5 = 2048


def _k1_body(x_ref, w_ref, b_ref, lo_ref, hi_ref):
    h = jnp.tanh(x_ref[...] * w_ref[...] + b_ref[...])
    lo_ref[...] = h[:, :HH]
    hi_ref[...] = h[:, HH:]


def _k1(x, lin0_W, lin0_b):
    grid = (pl.cdiv(N, BL1),)
    return pl.pallas_call(
        _k1_body,
        grid=grid,
        in_specs=[
            pl.BlockSpec((BL1, 1), lambda i: (i, 0)),
            pl.BlockSpec((1, H), lambda i: (0, 0)),
            pl.BlockSpec((1, H), lambda i: (0, 0)),
        ],
        out_specs=[
            pl.BlockSpec((BL1, HH), lambda i: (i, 0)),
            pl.BlockSpec((BL1, HH), lambda i: (i, 0)),
        ],
        out_shape=[jax.ShapeDtypeStruct((N, HH), jnp.float32),
                   jax.ShapeDtypeStruct((N, HH), jnp.float32)],
    )(x, lin0_W, lin0_b)


def _k3_body(alo_ref, ahi_ref, hlo_ref, hhi_ref, wrel_ref, brel_ref,
             wroot_ref, gamma_ref, beta_ref,
             zlo_ref, zhi_ref, s_ref, t_ref, st_acc):
    pid = pl.program_id(0)
    nb = pl.num_programs(0)
    agg = jnp.concatenate([alo_ref[...], ahi_ref[...]], axis=1)
    h = jnp.concatenate([hlo_ref[...], hhi_ref[...]], axis=1)
    z = jnp.maximum(_bdot(agg, wrel_ref[...]) + brel_ref[...]
                    + _bdot(h, wroot_ref[...]), 0.0)
    row = pid * BL3 + lax.broadcasted_iota(jnp.int32, (BL3, 1), 0)
    valid = (row < N).astype(jnp.float32)
    zm = z * valid
    zlo_ref[...] = z[:, :HH]
    zhi_ref[...] = z[:, HH:]
    ssum = jnp.sum(zm, axis=0).reshape(1, H)
    ssq = jnp.sum(zm * zm, axis=0).reshape(1, H)

    @pl.when(pid == 0)
    def _():
        st_acc[0:1, :] = ssum
        st_acc[1:2, :] = ssq

    @pl.when(pid > 0)
    def _():
        st_acc[0:1, :] = st_acc[0:1, :] + ssum
        st_acc[1:2, :] = st_acc[1:2, :] + ssq

    @pl.when(pid == nb - 1)
    def _():
        mu = st_acc[0:1, :] / float(N)
        var = st_acc[1:2, :] / float(N) - mu * mu
        s = gamma_ref[...] / jnp.sqrt(var + EPS)
        s_ref[...] = s
        t_ref[...] = beta_ref[...] - mu * s


def _k3(agg_lo, agg_hi, h_lo, h_hi, Wrel, brel, Wroot, gamma, beta):
    grid = (pl.cdiv(N, BL3),)
    full = lambda r, c: pl.BlockSpec((r, c), lambda i: (0, 0))
    blk = lambda: pl.BlockSpec((BL3, HH), lambda i: (i, 0))
    return pl.pallas_call(
        _k3_body,
        grid=grid,
        in_specs=[blk(), blk(), blk(), blk(),
                  full(H, H), full(1, H), full(H, H), full(1, H), full(1, H)],
        out_specs=[blk(), blk(), full(1, H), full(1, H)],
        out_shape=[jax.ShapeDtypeStruct((N, HH), jnp.float32),
                   jax.ShapeDtypeStruct((N, HH), jnp.float32),
                   jax.ShapeDtypeStruct((1, H), jnp.float32),
                   jax.ShapeDtypeStruct((1, H), jnp.float32)],
        scratch_shapes=[pltpu.VMEM((8, H), jnp.float32)],
    )(agg_lo, agg_hi, h_lo, h_hi, Wrel, brel, Wroot, gamma, beta)


def _k5_body(alo_ref, ahi_ref, zlo_ref, zhi_ref, ws_ref, batch_ref,
             s0_ref, t0_ref, wrel_ref, brel_ref, wroot_ref, gamma_ref, beta_ref,
             w1_ref, b1_ref, w2_ref, b2_ref,
             out_ref, st_acc, pool_acc, cnt_acc):
    pid = pl.program_id(0)
    nb = pl.num_programs(0)
    s0 = s0_ref[...]
    t0 = t0_ref[...]
    aggz = jnp.concatenate([alo_ref[...], ahi_ref[...]], axis=1)
    z0 = jnp.concatenate([zlo_ref[...], zhi_ref[...]], axis=1)
    h1 = z0 * s0 + t0
    agg1 = aggz * s0 + ws_ref[...] * t0
    z1 = jnp.maximum(_bdot(agg1, wrel_ref[...]) + brel_ref[...]
                     + _bdot(h1, wroot_ref[...]), 0.0)
    row = pid * BL5 + lax.broadcasted_iota(jnp.int32, (BL5, 1), 0)
    valid = (row < N).astype(jnp.float32)
    z1 = z1 * valid
    ssum = jnp.sum(z1, axis=0).reshape(1, H)
    ssq = jnp.sum(z1 * z1, axis=0).reshape(1, H)
    gids = lax.broadcasted_iota(jnp.int32, (BL5, G), 1)
    oh = (batch_ref[...] == gids).astype(jnp.float32) * valid
    pool = lax.dot_general(oh, z1, (((0,), (0,)), ((), ())),
                           preferred_element_type=jnp.float32,
                           precision=lax.Precision.HIGHEST)
    cnt = jnp.sum(oh, axis=0).reshape(1, G)

    @pl.when(pid == 0)
    def _():
        st_acc[0:1, :] = ssum
        st_acc[1:2, :] = ssq
        pool_acc[...] = pool
        cnt_acc[0:1, :] = cnt

    @pl.when(pid > 0)
    def _():
        st_acc[0:1, :] = st_acc[0:1, :] + ssum
        st_acc[1:2, :] = st_acc[1:2, :] + ssq
        pool_acc[...] = pool_acc[...] + pool
        cnt_acc[0:1, :] = cnt_acc[0:1, :] + cnt

    @pl.when(pid == nb - 1)
    def _():
        mu = st_acc[0:1, :] / float(N)
        var = st_acc[1:2, :] / float(N) - mu * mu
        s1 = gamma_ref[...] / jnp.sqrt(var + EPS)
        t1 = beta_ref[...] - mu * s1
        cnts = jnp.maximum(cnt_acc[0:1, :], 1.0).reshape(G, 1)
        gx = pool_acc[...] / cnts * s1 + t1
        hmid = jnp.maximum(_bdot(gx, w1_ref[...]) + b1_ref[...], 0.0)
        out_ref[...] = _bdot(hmid, w2_ref[...]) + b2_ref[...]


def _k5(agg_lo, agg_hi, z_lo, z_hi, wsum, batch2d, s0, t0,
        Wrel, brel, Wroot, gamma, beta, w1, b1, w2, b2):
    grid = (pl.cdiv(N, BL5),)
    full = lambda r, c: pl.BlockSpec((r, c), lambda i: (0, 0))
    blk = lambda w: pl.BlockSpec((BL5, w), lambda i: (i, 0))
    return pl.pallas_call(
        _k5_body,
        grid=grid,
        in_specs=[blk(HH), blk(HH), blk(HH), blk(HH), blk(1), blk(1),
                  full(1, H), full(1, H), full(H, H), full(1, H), full(H, H),
                  full(1, H), full(1, H), full(H, 64), full(1, 64),
                  full(64, 5), full(1, 5)],
        out_specs=pl.BlockSpec((G, 5), lambda i: (0, 0)),
        out_shape=jax.ShapeDtypeStruct((G, 5), jnp.float32),
        scratch_shapes=[pltpu.VMEM((8, H), jnp.float32),
                        pltpu.VMEM((G, H), jnp.float32),
                        pltpu.VMEM((8, G), jnp.float32)],
    )(agg_lo, agg_hi, z_lo, z_hi, wsum, batch2d, s0, t0,
      Wrel, brel, Wroot, gamma, beta, w1, b1, w2, b2)


def kernel(x, edge_index, edge_attr, batch, lin0_W, lin0_b, Wrel0, brel0,
           Wroot0, gamma0, beta0, Wrel1, brel1, Wroot1, gamma1, beta1,
           lin1_W, lin1_b, lin2_W, lin2_b):
    src = edge_index[0]
    dst = edge_index[1]
    w = edge_attr.reshape(-1)
    pad = EPAD - E
    padidx = jnp.arange(pad, dtype=jnp.int32) % N
    src_p = jnp.concatenate([src, padidx]).reshape(ROWS, 128)
    dst_p = jnp.concatenate([dst, padidx]).reshape(ROWS, 128)
    w_p = jnp.concatenate([w, jnp.zeros((pad,), jnp.float32)])

    h_lo, h_hi = _k1(x, lin0_W, lin0_b.reshape(1, H))
    agg_lo, agg_hi, wsum = _sc_agg_wsum(src_p, dst_p, w_p, h_lo, h_hi)
    z_lo, z_hi, s0, t0 = _k3(agg_lo, agg_hi, h_lo, h_hi,
                             Wrel0, brel0.reshape(1, H), Wroot0,
                             gamma0.reshape(1, H), beta0.reshape(1, H))
    a1_lo, a1_hi = _sc_agg(src_p, dst_p, w_p, z_lo, z_hi)
    pred = _k5(a1_lo, a1_hi, z_lo, z_hi, wsum.reshape(N, 1),
               batch.reshape(N, 1), s0, t0,
               Wrel1, brel1.reshape(1, H), Wroot1,
               gamma1.reshape(1, H), beta1.reshape(1, H),
               lin1_W, lin1_b.reshape(1, 64), lin2_W, lin2_b.reshape(1, 5))
    return pred
